# baseline (speedup 1.0000x reference)
import functools

import jax
import jax.numpy as jnp
from jax import lax
from jax.experimental import pallas as pl
from jax.experimental.pallas import tpu as pltpu
from jax.experimental.pallas import tpu_sc as plsc

_LANES = 16
_UNROLL = 8
_BS = 8192


def _build_sc_call(rows, n):
    info = plsc.get_sparse_core_info()
    num_cores, num_subcores = info.num_cores, info.num_subcores
    num_workers = num_cores * num_subcores
    rpw = rows // num_workers
    assert rpw * num_workers == rows
    nb = n // _BS
    chunks = n // _LANES
    bs_chunks = _BS // _LANES

    mesh = plsc.VectorSubcoreMesh(core_axis_name="c", subcore_axis_name="s")

    @functools.partial(
        pl.kernel,
        out_type=jax.ShapeDtypeStruct((rows, n), jnp.float32),
        mesh=mesh,
        scratch_types=[
            pltpu.VMEM((n,), jnp.float32),
            pltpu.VMEM((n,), jnp.float32),
            pltpu.VMEM((n,), jnp.float32),
            pltpu.VMEM_SHARED((_BS,), jnp.float32),
            pltpu.VMEM((_LANES,), jnp.float32),
            pltpu.SemaphoreType.DMA,
            pltpu.SemaphoreType.DMA,
            pltpu.SemaphoreType.DMA,
            pltpu.SemaphoreType.DMA,
        ],
        compiler_params=pltpu.CompilerParams(needs_layout_passes=False),
    )
    def sc_shift(items_hbm, pos_hbm, out_hbm, rb0, rb1, ob, zsh, posv,
                 si0, si1, so, sz):
        wid = lax.axis_index("s") * num_cores + lax.axis_index("c")
        base = wid * rpw

        pltpu.sync_copy(pos_hbm.at[pl.ds(base, rpw)], posv)
        idxv = (posv[...] * jnp.float32(n)).astype(jnp.int32)

        lanes = lax.iota(jnp.int32, _LANES)
        zero16 = jnp.zeros((_LANES,), jnp.float32)
        rbs, sins = (rb0, rb1), (si0, si1)

        @pl.when(lax.axis_index("s") == 0)
        def _():
            def _z0(j, c):
                ob[pl.ds(j * _LANES, _LANES)] = zero16
                return c
            lax.fori_loop(0, bs_chunks, _z0, 0)
            pltpu.sync_copy(ob.at[pl.ds(0, _BS)], zsh)
        plsc.subcore_barrier()

        def _nb_in(idx):
            return (jnp.int32(n) - idx + jnp.int32(_BS - 1)) >> 13

        def _fire_in(r, idx):
            def _f(j, c, r=r):
                pltpu.async_copy(
                    items_hbm.at[base + r,
                                 pl.ds(pl.multiple_of(j * _BS, _BS), _BS)],
                    rbs[r % 2].at[pl.ds(pl.multiple_of(j * _BS, _BS), _BS)],
                    sins[r % 2])
                return c
            lax.fori_loop(0, _nb_in(idx), _f, 0)

        def _wait_in(r, idx):
            def _w(j, c, r=r):
                pltpu.make_async_copy(
                    items_hbm.at[base + r, pl.ds(0, _BS)],
                    rbs[r % 2].at[pl.ds(0, _BS)], sins[r % 2]).wait()
                return c
            lax.fori_loop(0, _nb_in(idx), _w, 0)

        def _drain_out(r, zb):
            def _w(j, c, r=r):
                pltpu.make_async_copy(
                    ob.at[pl.ds(0, _BS)],
                    out_hbm.at[base + r, pl.ds(0, _BS)], so).wait()
                return c
            lax.fori_loop(zb, nb, _w, 0)

        def _drain_zeros(zb):
            def _w(j, c):
                pltpu.make_async_copy(
                    zsh, out_hbm.at[base, pl.ds(0, _BS)], sz).wait()
                return c
            lax.fori_loop(0, zb, _w, 0)

        _fire_in(0, idxv[0])

        for r in range(rpw):
            idx = idxv[r]
            zb = idx >> 13
            cz = idx >> 4

            if r + 1 < rpw:
                _fire_in(r + 1, idxv[r + 1])
            _wait_in(r, idx)
            rb = rbs[r % 2]

            def _fz(j, c, r=r):
                pltpu.async_copy(
                    zsh,
                    out_hbm.at[base + r,
                               pl.ds(pl.multiple_of(j * _BS, _BS), _BS)],
                    sz)
                return c
            lax.fori_loop(0, zb, _fz, 0)

            if r >= 1:
                _drain_out(r - 1, idxv[r - 1] >> 13)

            pz0 = zb * bs_chunks
            pzfull = (cz - pz0) // _UNROLL

            def _pzb(bnum, c, pz0=pz0):
                jb = pz0 + bnum * _UNROLL
                for u in range(_UNROLL):
                    ob[pl.ds(pl.multiple_of((jb + u) * _LANES, _LANES),
                             _LANES)] = zero16
                return c
            lax.fori_loop(0, pzfull, _pzb, 0)

            def _pz(j, c):
                ob[pl.ds(pl.multiple_of(j * _LANES, _LANES),
                         _LANES)] = zero16
                return c
            lax.fori_loop(pz0 + pzfull * _UNROLL, cz, _pz, 0)

            @pl.when(cz < chunks)
            def _(rb=rb, cz=cz, idx=idx):
                iv = lanes + (cz * _LANES - idx)
                m = iv >= 0
                v = plsc.load_gather(rb, [jnp.maximum(iv, 0)], mask=m)
                ob[pl.ds(pl.multiple_of(cz * _LANES, _LANES),
                         _LANES)] = jnp.where(m, v, 0.0)

            _GU = 2 * _UNROLL
            gstart = jnp.minimum(cz + 1, chunks)
            nfull = (chunks - gstart) // _GU

            def _loads(iv, rb=rb):
                return tuple(plsc.load_gather(rb, [iv + u * _LANES])
                             for u in range(_GU))

            def _stores(jb, vs):
                for u in range(_GU):
                    ob[pl.ds(pl.multiple_of((jb + u) * _LANES, _LANES),
                             _LANES)] = vs[u]

            iv0 = lanes + (gstart * _LANES - idx)

            @pl.when(nfull > 0)
            def _(gstart=gstart, nfull=nfull, iv0=iv0,
                  _loads=_loads, _stores=_stores):
                def _g(bnum, carry):
                    iv, prev = carry
                    iv_next = iv + _GU * _LANES
                    cur = _loads(iv_next)
                    _stores(gstart + bnum * _GU, prev)
                    return (iv_next, cur)

                _, last = lax.fori_loop(0, nfull - 1, _g,
                                        (iv0, _loads(iv0)))
                _stores(gstart + (nfull - 1) * _GU, last)

            def _gr(j, iv, rb=rb):
                v = plsc.load_gather(rb, [iv])
                ob[pl.ds(pl.multiple_of(j * _LANES, _LANES), _LANES)] = v
                return iv + _LANES

            lax.fori_loop(gstart + nfull * _GU, chunks, _gr,
                          iv0 + nfull * (_GU * _LANES))

            def _fo(j, c, r=r):
                pltpu.async_copy(
                    ob.at[pl.ds(pl.multiple_of(j * _BS, _BS), _BS)],
                    out_hbm.at[base + r,
                               pl.ds(pl.multiple_of(j * _BS, _BS), _BS)],
                    so)
                return c
            lax.fori_loop(zb, nb, _fo, 0)

        _drain_out(rpw - 1, idxv[rpw - 1] >> 13)
        for rr in range(rpw):
            _drain_zeros(idxv[rr] >> 13)

    return sc_shift


def kernel(items, positions, targets):
    b, nc, n = items.shape
    rows = b * nc
    items_f = items.reshape(rows, n)
    pos_f = positions.reshape(rows)
    out = _build_sc_call(rows, n)(items_f, pos_f)
    return out.reshape(b, nc, n)


# speedup vs baseline: 1.0644x; 1.0644x over previous
import functools

import jax
import jax.numpy as jnp
from jax import lax
from jax.experimental import pallas as pl
from jax.experimental.pallas import tpu as pltpu
from jax.experimental.pallas import tpu_sc as plsc

_LANES = 16
_UNROLL = 8
_BS = 4096
_BSH = _BS.bit_length() - 1


def _build_sc_call(rows, n):
    info = plsc.get_sparse_core_info()
    num_cores, num_subcores = info.num_cores, info.num_subcores
    num_workers = num_cores * num_subcores
    rpw = rows // num_workers
    assert rpw * num_workers == rows
    nb = n // _BS
    chunks = n // _LANES
    bs_chunks = _BS // _LANES

    mesh = plsc.VectorSubcoreMesh(core_axis_name="c", subcore_axis_name="s")

    @functools.partial(
        pl.kernel,
        out_type=jax.ShapeDtypeStruct((rows, n), jnp.float32),
        mesh=mesh,
        scratch_types=[
            pltpu.VMEM((n,), jnp.float32),
            pltpu.VMEM((n,), jnp.float32),
            pltpu.VMEM((n,), jnp.float32),
            pltpu.VMEM_SHARED((_BS,), jnp.float32),
            pltpu.VMEM((_LANES,), jnp.float32),
            pltpu.SemaphoreType.DMA,
            pltpu.SemaphoreType.DMA,
            pltpu.SemaphoreType.DMA,
            pltpu.SemaphoreType.DMA,
        ],
        compiler_params=pltpu.CompilerParams(needs_layout_passes=False),
    )
    def sc_shift(items_hbm, pos_hbm, out_hbm, rb0, rb1, ob, zsh, posv,
                 si0, si1, so, sz):
        wid = lax.axis_index("s") * num_cores + lax.axis_index("c")
        base = wid * rpw

        pltpu.sync_copy(pos_hbm.at[pl.ds(base, rpw)], posv)
        idxv = (posv[...] * jnp.float32(n)).astype(jnp.int32)

        lanes = lax.iota(jnp.int32, _LANES)
        zero16 = jnp.zeros((_LANES,), jnp.float32)
        rbs, sins = (rb0, rb1), (si0, si1)

        @pl.when(lax.axis_index("s") == 0)
        def _():
            def _z0(j, c):
                ob[pl.ds(j * _LANES, _LANES)] = zero16
                return c
            lax.fori_loop(0, bs_chunks, _z0, 0)
            pltpu.sync_copy(ob.at[pl.ds(0, _BS)], zsh)
        plsc.subcore_barrier()

        def _nb_in(idx):
            return (jnp.int32(n) - idx + jnp.int32(_BS - 1)) >> _BSH

        def _fire_in(r, idx):
            def _f(j, c, r=r):
                pltpu.async_copy(
                    items_hbm.at[base + r,
                                 pl.ds(pl.multiple_of(j * _BS, _BS), _BS)],
                    rbs[r % 2].at[pl.ds(pl.multiple_of(j * _BS, _BS), _BS)],
                    sins[r % 2])
                return c
            lax.fori_loop(0, _nb_in(idx), _f, 0)

        def _wait_in(r, idx):
            def _w(j, c, r=r):
                pltpu.make_async_copy(
                    items_hbm.at[base + r, pl.ds(0, _BS)],
                    rbs[r % 2].at[pl.ds(0, _BS)], sins[r % 2]).wait()
                return c
            lax.fori_loop(0, _nb_in(idx), _w, 0)

        def _drain_out(r, zb):
            def _w(j, c, r=r):
                pltpu.make_async_copy(
                    ob.at[pl.ds(0, _BS)],
                    out_hbm.at[base + r, pl.ds(0, _BS)], so).wait()
                return c
            lax.fori_loop(zb, nb, _w, 0)

        def _drain_zeros(zb):
            def _w(j, c):
                pltpu.make_async_copy(
                    zsh, out_hbm.at[base, pl.ds(0, _BS)], sz).wait()
                return c
            lax.fori_loop(0, zb, _w, 0)

        _fire_in(0, idxv[0])

        for r in range(rpw):
            idx = idxv[r]
            zb = idx >> _BSH
            cz = idx >> 4

            if r + 1 < rpw:
                _fire_in(r + 1, idxv[r + 1])
            _wait_in(r, idx)
            rb = rbs[r % 2]

            def _fz(j, c, r=r):
                pltpu.async_copy(
                    zsh,
                    out_hbm.at[base + r,
                               pl.ds(pl.multiple_of(j * _BS, _BS), _BS)],
                    sz)
                return c
            lax.fori_loop(0, zb, _fz, 0)

            if r >= 1:
                _drain_out(r - 1, idxv[r - 1] >> _BSH)

            pz0 = zb * bs_chunks
            pzfull = (cz - pz0) // _UNROLL

            def _pzb(bnum, c, pz0=pz0):
                jb = pz0 + bnum * _UNROLL
                for u in range(_UNROLL):
                    ob[pl.ds(pl.multiple_of((jb + u) * _LANES, _LANES),
                             _LANES)] = zero16
                return c
            lax.fori_loop(0, pzfull, _pzb, 0)

            def _pz(j, c):
                ob[pl.ds(pl.multiple_of(j * _LANES, _LANES),
                         _LANES)] = zero16
                return c
            lax.fori_loop(pz0 + pzfull * _UNROLL, cz, _pz, 0)

            @pl.when(cz < chunks)
            def _(rb=rb, cz=cz, idx=idx):
                iv = lanes + (cz * _LANES - idx)
                m = iv >= 0
                v = plsc.load_gather(rb, [jnp.maximum(iv, 0)], mask=m)
                ob[pl.ds(pl.multiple_of(cz * _LANES, _LANES),
                         _LANES)] = jnp.where(m, v, 0.0)

            gstart = jnp.minimum(cz + 1, chunks)
            nfull = (chunks - gstart) // _UNROLL

            def _loads(iv, rb=rb):
                return tuple(plsc.load_gather(rb, [iv + u * _LANES])
                             for u in range(_UNROLL))

            def _stores(jb, vs):
                for u in range(_UNROLL):
                    ob[pl.ds(pl.multiple_of((jb + u) * _LANES, _LANES),
                             _LANES)] = vs[u]

            iv0 = lanes + (gstart * _LANES - idx)

            @pl.when(nfull > 0)
            def _(gstart=gstart, nfull=nfull, iv0=iv0,
                  _loads=_loads, _stores=_stores):
                def _g(bnum, carry):
                    iv, prev = carry
                    iv_next = iv + _UNROLL * _LANES
                    cur = _loads(iv_next)
                    _stores(gstart + bnum * _UNROLL, prev)
                    return (iv_next, cur)

                _, last = lax.fori_loop(0, nfull - 1, _g,
                                        (iv0, _loads(iv0)))
                _stores(gstart + (nfull - 1) * _UNROLL, last)

            def _gr(j, iv, rb=rb):
                v = plsc.load_gather(rb, [iv])
                ob[pl.ds(pl.multiple_of(j * _LANES, _LANES), _LANES)] = v
                return iv + _LANES

            lax.fori_loop(gstart + nfull * _UNROLL, chunks, _gr,
                          iv0 + nfull * (_UNROLL * _LANES))

            def _fo(j, c, r=r):
                pltpu.async_copy(
                    ob.at[pl.ds(pl.multiple_of(j * _BS, _BS), _BS)],
                    out_hbm.at[base + r,
                               pl.ds(pl.multiple_of(j * _BS, _BS), _BS)],
                    so)
                return c
            lax.fori_loop(zb, nb, _fo, 0)

        _drain_out(rpw - 1, idxv[rpw - 1] >> _BSH)
        for rr in range(rpw):
            _drain_zeros(idxv[rr] >> _BSH)

    return sc_shift


def kernel(items, positions, targets):
    b, nc, n = items.shape
    rows = b * nc
    items_f = items.reshape(rows, n)
    pos_f = positions.reshape(rows)
    out = _build_sc_call(rows, n)(items_f, pos_f)
    return out.reshape(b, nc, n)


# speedup vs baseline: 1.0803x; 1.0150x over previous
import functools

import jax
import jax.numpy as jnp
from jax import lax
from jax.experimental import pallas as pl
from jax.experimental.pallas import tpu as pltpu
from jax.experimental.pallas import tpu_sc as plsc

_LANES = 16
_UNROLL = 8
_BS = 2048
_BSH = _BS.bit_length() - 1


def _build_sc_call(rows, n):
    info = plsc.get_sparse_core_info()
    num_cores, num_subcores = info.num_cores, info.num_subcores
    num_workers = num_cores * num_subcores
    rpw = rows // num_workers
    assert rpw * num_workers == rows
    nb = n // _BS
    chunks = n // _LANES
    bs_chunks = _BS // _LANES

    mesh = plsc.VectorSubcoreMesh(core_axis_name="c", subcore_axis_name="s")

    @functools.partial(
        pl.kernel,
        out_type=jax.ShapeDtypeStruct((rows, n), jnp.float32),
        mesh=mesh,
        scratch_types=[
            pltpu.VMEM((n,), jnp.float32),
            pltpu.VMEM((n,), jnp.float32),
            pltpu.VMEM((n,), jnp.float32),
            pltpu.VMEM_SHARED((_BS,), jnp.float32),
            pltpu.VMEM((_LANES,), jnp.float32),
            pltpu.SemaphoreType.DMA,
            pltpu.SemaphoreType.DMA,
            pltpu.SemaphoreType.DMA,
            pltpu.SemaphoreType.DMA,
        ],
        compiler_params=pltpu.CompilerParams(needs_layout_passes=False),
    )
    def sc_shift(items_hbm, pos_hbm, out_hbm, rb0, rb1, ob, zsh, posv,
                 si0, si1, so, sz):
        wid = lax.axis_index("s") * num_cores + lax.axis_index("c")
        base = wid * rpw

        pltpu.sync_copy(pos_hbm.at[pl.ds(base, rpw)], posv)
        idxv = (posv[...] * jnp.float32(n)).astype(jnp.int32)

        lanes = lax.iota(jnp.int32, _LANES)
        zero16 = jnp.zeros((_LANES,), jnp.float32)
        rbs, sins = (rb0, rb1), (si0, si1)

        @pl.when(lax.axis_index("s") == 0)
        def _():
            def _z0(j, c):
                ob[pl.ds(j * _LANES, _LANES)] = zero16
                return c
            lax.fori_loop(0, bs_chunks, _z0, 0)
            pltpu.sync_copy(ob.at[pl.ds(0, _BS)], zsh)
        plsc.subcore_barrier()

        def _nb_in(idx):
            return (jnp.int32(n) - idx + jnp.int32(_BS - 1)) >> _BSH

        def _fire_in(r, idx):
            def _f(j, c, r=r):
                pltpu.async_copy(
                    items_hbm.at[base + r,
                                 pl.ds(pl.multiple_of(j * _BS, _BS), _BS)],
                    rbs[r % 2].at[pl.ds(pl.multiple_of(j * _BS, _BS), _BS)],
                    sins[r % 2])
                return c
            lax.fori_loop(0, _nb_in(idx), _f, 0)

        def _wait_in(r, idx):
            def _w(j, c, r=r):
                pltpu.make_async_copy(
                    items_hbm.at[base + r, pl.ds(0, _BS)],
                    rbs[r % 2].at[pl.ds(0, _BS)], sins[r % 2]).wait()
                return c
            lax.fori_loop(0, _nb_in(idx), _w, 0)

        def _drain_out(r, zb):
            def _w(j, c, r=r):
                pltpu.make_async_copy(
                    ob.at[pl.ds(0, _BS)],
                    out_hbm.at[base + r, pl.ds(0, _BS)], so).wait()
                return c
            lax.fori_loop(zb, nb, _w, 0)

        def _drain_zeros(zb):
            def _w(j, c):
                pltpu.make_async_copy(
                    zsh, out_hbm.at[base, pl.ds(0, _BS)], sz).wait()
                return c
            lax.fori_loop(0, zb, _w, 0)

        _fire_in(0, idxv[0])

        for r in range(rpw):
            idx = idxv[r]
            zb = idx >> _BSH
            cz = idx >> 4

            if r + 1 < rpw:
                _fire_in(r + 1, idxv[r + 1])
            _wait_in(r, idx)
            rb = rbs[r % 2]

            def _fz(j, c, r=r):
                pltpu.async_copy(
                    zsh,
                    out_hbm.at[base + r,
                               pl.ds(pl.multiple_of(j * _BS, _BS), _BS)],
                    sz)
                return c
            lax.fori_loop(0, zb, _fz, 0)

            if r >= 1:
                _drain_out(r - 1, idxv[r - 1] >> _BSH)

            pz0 = zb * bs_chunks
            pzfull = (cz - pz0) // _UNROLL

            def _pzb(bnum, c, pz0=pz0):
                jb = pz0 + bnum * _UNROLL
                for u in range(_UNROLL):
                    ob[pl.ds(pl.multiple_of((jb + u) * _LANES, _LANES),
                             _LANES)] = zero16
                return c
            lax.fori_loop(0, pzfull, _pzb, 0)

            def _pz(j, c):
                ob[pl.ds(pl.multiple_of(j * _LANES, _LANES),
                         _LANES)] = zero16
                return c
            lax.fori_loop(pz0 + pzfull * _UNROLL, cz, _pz, 0)

            @pl.when(cz < chunks)
            def _(rb=rb, cz=cz, idx=idx):
                iv = lanes + (cz * _LANES - idx)
                m = iv >= 0
                v = plsc.load_gather(rb, [jnp.maximum(iv, 0)], mask=m)
                ob[pl.ds(pl.multiple_of(cz * _LANES, _LANES),
                         _LANES)] = jnp.where(m, v, 0.0)

            gstart = jnp.minimum(cz + 1, chunks)
            nfull = (chunks - gstart) // _UNROLL

            def _loads(iv, rb=rb):
                return tuple(plsc.load_gather(rb, [iv + u * _LANES])
                             for u in range(_UNROLL))

            def _stores(jb, vs):
                for u in range(_UNROLL):
                    ob[pl.ds(pl.multiple_of((jb + u) * _LANES, _LANES),
                             _LANES)] = vs[u]

            iv0 = lanes + (gstart * _LANES - idx)

            @pl.when(nfull > 0)
            def _(gstart=gstart, nfull=nfull, iv0=iv0,
                  _loads=_loads, _stores=_stores):
                def _g(bnum, carry):
                    iv, prev = carry
                    iv_next = iv + _UNROLL * _LANES
                    cur = _loads(iv_next)
                    _stores(gstart + bnum * _UNROLL, prev)
                    return (iv_next, cur)

                _, last = lax.fori_loop(0, nfull - 1, _g,
                                        (iv0, _loads(iv0)))
                _stores(gstart + (nfull - 1) * _UNROLL, last)

            def _gr(j, iv, rb=rb):
                v = plsc.load_gather(rb, [iv])
                ob[pl.ds(pl.multiple_of(j * _LANES, _LANES), _LANES)] = v
                return iv + _LANES

            lax.fori_loop(gstart + nfull * _UNROLL, chunks, _gr,
                          iv0 + nfull * (_UNROLL * _LANES))

            def _fo(j, c, r=r):
                pltpu.async_copy(
                    ob.at[pl.ds(pl.multiple_of(j * _BS, _BS), _BS)],
                    out_hbm.at[base + r,
                               pl.ds(pl.multiple_of(j * _BS, _BS), _BS)],
                    so)
                return c
            lax.fori_loop(zb, nb, _fo, 0)

        _drain_out(rpw - 1, idxv[rpw - 1] >> _BSH)
        for rr in range(rpw):
            _drain_zeros(idxv[rr] >> _BSH)

    return sc_shift


def kernel(items, positions, targets):
    b, nc, n = items.shape
    rows = b * nc
    items_f = items.reshape(rows, n)
    pos_f = positions.reshape(rows)
    out = _build_sc_call(rows, n)(items_f, pos_f)
    return out.reshape(b, nc, n)


# speedup vs baseline: 1.0861x; 1.0053x over previous
import functools

import jax
import jax.numpy as jnp
from jax import lax
from jax.experimental import pallas as pl
from jax.experimental.pallas import tpu as pltpu
from jax.experimental.pallas import tpu_sc as plsc

_LANES = 16
_UNROLL = 8
_BS = 1024
_BSH = _BS.bit_length() - 1


def _build_sc_call(rows, n):
    info = plsc.get_sparse_core_info()
    num_cores, num_subcores = info.num_cores, info.num_subcores
    num_workers = num_cores * num_subcores
    rpw = rows // num_workers
    assert rpw * num_workers == rows
    nb = n // _BS
    chunks = n // _LANES
    bs_chunks = _BS // _LANES

    mesh = plsc.VectorSubcoreMesh(core_axis_name="c", subcore_axis_name="s")

    @functools.partial(
        pl.kernel,
        out_type=jax.ShapeDtypeStruct((rows, n), jnp.float32),
        mesh=mesh,
        scratch_types=[
            pltpu.VMEM((n,), jnp.float32),
            pltpu.VMEM((n,), jnp.float32),
            pltpu.VMEM((n,), jnp.float32),
            pltpu.VMEM_SHARED((_BS,), jnp.float32),
            pltpu.VMEM((_LANES,), jnp.float32),
            pltpu.SemaphoreType.DMA,
            pltpu.SemaphoreType.DMA,
            pltpu.SemaphoreType.DMA,
            pltpu.SemaphoreType.DMA,
        ],
        compiler_params=pltpu.CompilerParams(needs_layout_passes=False),
    )
    def sc_shift(items_hbm, pos_hbm, out_hbm, rb0, rb1, ob, zsh, posv,
                 si0, si1, so, sz):
        wid = lax.axis_index("s") * num_cores + lax.axis_index("c")
        base = wid * rpw

        pltpu.sync_copy(pos_hbm.at[pl.ds(base, rpw)], posv)
        idxv = (posv[...] * jnp.float32(n)).astype(jnp.int32)

        lanes = lax.iota(jnp.int32, _LANES)
        zero16 = jnp.zeros((_LANES,), jnp.float32)
        rbs, sins = (rb0, rb1), (si0, si1)

        @pl.when(lax.axis_index("s") == 0)
        def _():
            def _z0(j, c):
                ob[pl.ds(j * _LANES, _LANES)] = zero16
                return c
            lax.fori_loop(0, bs_chunks, _z0, 0)
            pltpu.sync_copy(ob.at[pl.ds(0, _BS)], zsh)
        plsc.subcore_barrier()

        def _nb_in(idx):
            return (jnp.int32(n) - idx + jnp.int32(_BS - 1)) >> _BSH

        def _fire_in(r, idx):
            def _f(j, c, r=r):
                pltpu.async_copy(
                    items_hbm.at[base + r,
                                 pl.ds(pl.multiple_of(j * _BS, _BS), _BS)],
                    rbs[r % 2].at[pl.ds(pl.multiple_of(j * _BS, _BS), _BS)],
                    sins[r % 2])
                return c
            lax.fori_loop(0, _nb_in(idx), _f, 0)

        def _wait_in(r, idx):
            def _w(j, c, r=r):
                pltpu.make_async_copy(
                    items_hbm.at[base + r, pl.ds(0, _BS)],
                    rbs[r % 2].at[pl.ds(0, _BS)], sins[r % 2]).wait()
                return c
            lax.fori_loop(0, _nb_in(idx), _w, 0)

        def _drain_out(r, zb):
            def _w(j, c, r=r):
                pltpu.make_async_copy(
                    ob.at[pl.ds(0, _BS)],
                    out_hbm.at[base + r, pl.ds(0, _BS)], so).wait()
                return c
            lax.fori_loop(zb, nb, _w, 0)

        def _drain_zeros(zb):
            def _w(j, c):
                pltpu.make_async_copy(
                    zsh, out_hbm.at[base, pl.ds(0, _BS)], sz).wait()
                return c
            lax.fori_loop(0, zb, _w, 0)

        _fire_in(0, idxv[0])

        for r in range(rpw):
            idx = idxv[r]
            zb = idx >> _BSH
            cz = idx >> 4

            if r + 1 < rpw:
                _fire_in(r + 1, idxv[r + 1])
            _wait_in(r, idx)
            rb = rbs[r % 2]

            def _fz(j, c, r=r):
                pltpu.async_copy(
                    zsh,
                    out_hbm.at[base + r,
                               pl.ds(pl.multiple_of(j * _BS, _BS), _BS)],
                    sz)
                return c
            lax.fori_loop(0, zb, _fz, 0)

            if r >= 1:
                _drain_out(r - 1, idxv[r - 1] >> _BSH)

            pz0 = zb * bs_chunks
            pzfull = (cz - pz0) // _UNROLL

            def _pzb(bnum, c, pz0=pz0):
                jb = pz0 + bnum * _UNROLL
                for u in range(_UNROLL):
                    ob[pl.ds(pl.multiple_of((jb + u) * _LANES, _LANES),
                             _LANES)] = zero16
                return c
            lax.fori_loop(0, pzfull, _pzb, 0)

            def _pz(j, c):
                ob[pl.ds(pl.multiple_of(j * _LANES, _LANES),
                         _LANES)] = zero16
                return c
            lax.fori_loop(pz0 + pzfull * _UNROLL, cz, _pz, 0)

            @pl.when(cz < chunks)
            def _(rb=rb, cz=cz, idx=idx):
                iv = lanes + (cz * _LANES - idx)
                m = iv >= 0
                v = plsc.load_gather(rb, [jnp.maximum(iv, 0)], mask=m)
                ob[pl.ds(pl.multiple_of(cz * _LANES, _LANES),
                         _LANES)] = jnp.where(m, v, 0.0)

            gstart = jnp.minimum(cz + 1, chunks)
            nfull = (chunks - gstart) // _UNROLL

            def _loads(iv, rb=rb):
                return tuple(plsc.load_gather(rb, [iv + u * _LANES])
                             for u in range(_UNROLL))

            def _stores(jb, vs):
                for u in range(_UNROLL):
                    ob[pl.ds(pl.multiple_of((jb + u) * _LANES, _LANES),
                             _LANES)] = vs[u]

            iv0 = lanes + (gstart * _LANES - idx)

            @pl.when(nfull > 0)
            def _(gstart=gstart, nfull=nfull, iv0=iv0,
                  _loads=_loads, _stores=_stores):
                def _g(bnum, carry):
                    iv, prev = carry
                    iv_next = iv + _UNROLL * _LANES
                    cur = _loads(iv_next)
                    _stores(gstart + bnum * _UNROLL, prev)
                    return (iv_next, cur)

                _, last = lax.fori_loop(0, nfull - 1, _g,
                                        (iv0, _loads(iv0)))
                _stores(gstart + (nfull - 1) * _UNROLL, last)

            def _gr(j, iv, rb=rb):
                v = plsc.load_gather(rb, [iv])
                ob[pl.ds(pl.multiple_of(j * _LANES, _LANES), _LANES)] = v
                return iv + _LANES

            lax.fori_loop(gstart + nfull * _UNROLL, chunks, _gr,
                          iv0 + nfull * (_UNROLL * _LANES))

            def _fo(j, c, r=r):
                pltpu.async_copy(
                    ob.at[pl.ds(pl.multiple_of(j * _BS, _BS), _BS)],
                    out_hbm.at[base + r,
                               pl.ds(pl.multiple_of(j * _BS, _BS), _BS)],
                    so)
                return c
            lax.fori_loop(zb, nb, _fo, 0)

        _drain_out(rpw - 1, idxv[rpw - 1] >> _BSH)
        for rr in range(rpw):
            _drain_zeros(idxv[rr] >> _BSH)

    return sc_shift


def kernel(items, positions, targets):
    b, nc, n = items.shape
    rows = b * nc
    items_f = items.reshape(rows, n)
    pos_f = positions.reshape(rows)
    out = _build_sc_call(rows, n)(items_f, pos_f)
    return out.reshape(b, nc, n)


# speedup vs baseline: 1.0866x; 1.0005x over previous
import functools

import jax
import jax.numpy as jnp
from jax import lax
from jax.experimental import pallas as pl
from jax.experimental.pallas import tpu as pltpu
from jax.experimental.pallas import tpu_sc as plsc

_LANES = 16
_UNROLL = 8
_BS = 2048
_BSH = _BS.bit_length() - 1


def _build_sc_call(rows, n):
    info = plsc.get_sparse_core_info()
    num_cores, num_subcores = info.num_cores, info.num_subcores
    num_workers = num_cores * num_subcores
    rpw = rows // num_workers
    assert rpw * num_workers == rows
    nb = n // _BS
    chunks = n // _LANES
    bs_chunks = _BS // _LANES

    mesh = plsc.VectorSubcoreMesh(core_axis_name="c", subcore_axis_name="s")

    @functools.partial(
        pl.kernel,
        out_type=jax.ShapeDtypeStruct((rows, n), jnp.float32),
        mesh=mesh,
        scratch_types=[
            pltpu.VMEM((n,), jnp.float32),
            pltpu.VMEM((n,), jnp.float32),
            pltpu.VMEM((n,), jnp.float32),
            pltpu.VMEM_SHARED((_BS,), jnp.float32),
            pltpu.VMEM((_LANES,), jnp.float32),
            pltpu.SemaphoreType.DMA,
            pltpu.SemaphoreType.DMA,
            pltpu.SemaphoreType.DMA,
            pltpu.SemaphoreType.DMA,
        ],
        compiler_params=pltpu.CompilerParams(needs_layout_passes=False),
    )
    def sc_shift(items_hbm, pos_hbm, out_hbm, rb0, rb1, ob, zsh, posv,
                 si0, si1, so, sz):
        wid = lax.axis_index("s") * num_cores + lax.axis_index("c")
        base = wid * rpw

        pltpu.sync_copy(pos_hbm.at[pl.ds(base, rpw)], posv)
        idxv = (posv[...] * jnp.float32(n)).astype(jnp.int32)

        lanes = lax.iota(jnp.int32, _LANES)
        zero16 = jnp.zeros((_LANES,), jnp.float32)
        rbs, sins = (rb0, rb1), (si0, si1)

        def _z0(j, c):
            ob[pl.ds(j * _LANES, _LANES)] = zero16
            return c
        lax.fori_loop(0, bs_chunks, _z0, 0)
        pltpu.sync_copy(ob.at[pl.ds(0, _BS)], zsh)

        def _nb_in(idx):
            return (jnp.int32(n) - idx + jnp.int32(_BS - 1)) >> _BSH

        def _fire_in(r, idx):
            def _f(j, c, r=r):
                pltpu.async_copy(
                    items_hbm.at[base + r,
                                 pl.ds(pl.multiple_of(j * _BS, _BS), _BS)],
                    rbs[r % 2].at[pl.ds(pl.multiple_of(j * _BS, _BS), _BS)],
                    sins[r % 2])
                return c
            lax.fori_loop(0, _nb_in(idx), _f, 0)

        def _wait_in(r, idx):
            def _w(j, c, r=r):
                pltpu.make_async_copy(
                    items_hbm.at[base + r, pl.ds(0, _BS)],
                    rbs[r % 2].at[pl.ds(0, _BS)], sins[r % 2]).wait()
                return c
            lax.fori_loop(0, _nb_in(idx), _w, 0)

        def _drain_out(r, zb):
            def _w(j, c, r=r):
                pltpu.make_async_copy(
                    ob.at[pl.ds(0, _BS)],
                    out_hbm.at[base + r, pl.ds(0, _BS)], so).wait()
                return c
            lax.fori_loop(zb, nb, _w, 0)

        def _drain_zeros(zb):
            def _w(j, c):
                pltpu.make_async_copy(
                    zsh, out_hbm.at[base, pl.ds(0, _BS)], sz).wait()
                return c
            lax.fori_loop(0, zb, _w, 0)

        _fire_in(0, idxv[0])

        for r in range(rpw):
            idx = idxv[r]
            zb = idx >> _BSH
            cz = idx >> 4

            if r + 1 < rpw:
                _fire_in(r + 1, idxv[r + 1])
            _wait_in(r, idx)
            rb = rbs[r % 2]

            def _fz(j, c, r=r):
                pltpu.async_copy(
                    zsh,
                    out_hbm.at[base + r,
                               pl.ds(pl.multiple_of(j * _BS, _BS), _BS)],
                    sz)
                return c
            lax.fori_loop(0, zb, _fz, 0)

            if r >= 1:
                _drain_out(r - 1, idxv[r - 1] >> _BSH)
                _drain_zeros(idxv[r - 1] >> _BSH)

            pz0 = zb * bs_chunks
            pzfull = (cz - pz0) // _UNROLL

            def _pzb(bnum, c, pz0=pz0):
                jb = pz0 + bnum * _UNROLL
                for u in range(_UNROLL):
                    ob[pl.ds(pl.multiple_of((jb + u) * _LANES, _LANES),
                             _LANES)] = zero16
                return c
            lax.fori_loop(0, pzfull, _pzb, 0)

            def _pz(j, c):
                ob[pl.ds(pl.multiple_of(j * _LANES, _LANES),
                         _LANES)] = zero16
                return c
            lax.fori_loop(pz0 + pzfull * _UNROLL, cz, _pz, 0)

            @pl.when(cz < chunks)
            def _(rb=rb, cz=cz, idx=idx):
                iv = lanes + (cz * _LANES - idx)
                m = iv >= 0
                v = plsc.load_gather(rb, [jnp.maximum(iv, 0)], mask=m)
                ob[pl.ds(pl.multiple_of(cz * _LANES, _LANES),
                         _LANES)] = jnp.where(m, v, 0.0)

            gstart = jnp.minimum(cz + 1, chunks)
            nfull = (chunks - gstart) // _UNROLL

            def _loads(iv, rb=rb):
                return tuple(plsc.load_gather(rb, [iv + u * _LANES])
                             for u in range(_UNROLL))

            def _stores(jb, vs):
                for u in range(_UNROLL):
                    ob[pl.ds(pl.multiple_of((jb + u) * _LANES, _LANES),
                             _LANES)] = vs[u]

            iv0 = lanes + (gstart * _LANES - idx)

            @pl.when(nfull > 0)
            def _(gstart=gstart, nfull=nfull, iv0=iv0,
                  _loads=_loads, _stores=_stores):
                def _g(bnum, carry):
                    iv, prev = carry
                    iv_next = iv + _UNROLL * _LANES
                    cur = _loads(iv_next)
                    _stores(gstart + bnum * _UNROLL, prev)
                    return (iv_next, cur)

                _, last = lax.fori_loop(0, nfull - 1, _g,
                                        (iv0, _loads(iv0)))
                _stores(gstart + (nfull - 1) * _UNROLL, last)

            def _gr(j, iv, rb=rb):
                v = plsc.load_gather(rb, [iv])
                ob[pl.ds(pl.multiple_of(j * _LANES, _LANES), _LANES)] = v
                return iv + _LANES

            lax.fori_loop(gstart + nfull * _UNROLL, chunks, _gr,
                          iv0 + nfull * (_UNROLL * _LANES))

            def _fo(j, c, r=r):
                pltpu.async_copy(
                    ob.at[pl.ds(pl.multiple_of(j * _BS, _BS), _BS)],
                    out_hbm.at[base + r,
                               pl.ds(pl.multiple_of(j * _BS, _BS), _BS)],
                    so)
                return c
            lax.fori_loop(zb, nb, _fo, 0)

        _drain_out(rpw - 1, idxv[rpw - 1] >> _BSH)
        _drain_zeros(idxv[rpw - 1] >> _BSH)

    return sc_shift


def kernel(items, positions, targets):
    b, nc, n = items.shape
    rows = b * nc
    items_f = items.reshape(rows, n)
    pos_f = positions.reshape(rows)
    out = _build_sc_call(rows, n)(items_f, pos_f)
    return out.reshape(b, nc, n)
